# baseline (speedup 1.0000x reference)
import jax
import jax.numpy as jnp
from jax.experimental import pallas as pl

N, D, E, H = 2048, 1024, 8, 4096
BLK = 256
HB = 1024
NB = N // BLK
NH = H // HB


def _ffn_kernel(x_ref, disp_ref, comb_ref, wg_ref, wv_ref, wo_ref, scale_ref, out_ref):
    e = pl.program_id(0)
    h = pl.program_id(1)
    i = pl.program_id(2)

    x = x_ref[...]
    wg = wg_ref[0]
    wv = wv_ref[0]
    wo = wo_ref[0]

    gate = jax.lax.dot_general(x, wg, (((1,), (1,)), ((), ())),
                               preferred_element_type=jnp.float32)
    gate = gate * 0.5 * (1.0 + jax.lax.erf(gate * 0.7071067811865476))
    value = jax.lax.dot_general(x, wv, (((1,), (1,)), ((), ())),
                                preferred_element_type=jnp.float32)
    hidden = (gate * value).astype(jnp.bfloat16)
    part = jax.lax.dot_general(hidden, wo, (((1,), (1,)), ((), ())),
                               preferred_element_type=jnp.float32)

    cols = jax.lax.broadcasted_iota(jnp.int32, (BLK, E), 1) == e
    dcol = jnp.sum(jnp.where(cols, disp_ref[...], 0.0), axis=1, keepdims=True)
    ccol = jnp.sum(jnp.where(cols, comb_ref[...], 0.0), axis=1, keepdims=True)
    srow = jax.lax.broadcasted_iota(jnp.int32, (1, E), 1) == e
    scale_e = jnp.sum(jnp.where(srow, scale_ref[...], 0.0))
    coef = jnp.where(dcol > 0, ccol * scale_e, 0.0)
    contrib = part * coef

    rows = pl.ds(i * BLK, BLK)

    @pl.when(jnp.logical_and(e == 0, h == 0))
    def _init():
        out_ref[rows, :] = contrib

    @pl.when(jnp.logical_not(jnp.logical_and(e == 0, h == 0)))
    def _acc():
        out_ref[rows, :] += contrib


def kernel(tokens, dispatch_weights, combine_weights, Wg, Wv, Wo, scale):
    b, n, d = tokens.shape
    flat = tokens.reshape(n, d).astype(jnp.bfloat16)
    disp = dispatch_weights.reshape(n, E)
    comb = combine_weights.reshape(n, E)
    Wg = Wg.astype(jnp.bfloat16)
    Wv = Wv.astype(jnp.bfloat16)
    Wo = Wo.astype(jnp.bfloat16)

    out = pl.pallas_call(
        _ffn_kernel,
        grid=(E, NH, NB),
        in_specs=[
            pl.BlockSpec((BLK, D), lambda e, h, i: (i, 0)),
            pl.BlockSpec((BLK, E), lambda e, h, i: (i, 0)),
            pl.BlockSpec((BLK, E), lambda e, h, i: (i, 0)),
            pl.BlockSpec((1, HB, D), lambda e, h, i: (e, h, 0)),
            pl.BlockSpec((1, HB, D), lambda e, h, i: (e, h, 0)),
            pl.BlockSpec((1, D, HB), lambda e, h, i: (e, 0, h)),
            pl.BlockSpec((1, E), lambda e, h, i: (0, 0)),
        ],
        out_specs=pl.BlockSpec((N, D), lambda e, h, i: (0, 0)),
        out_shape=jax.ShapeDtypeStruct((N, D), jnp.float32),
    )(flat, disp, comb, Wg, Wv, Wo, scale.reshape(1, E))
    return out.reshape(b, n, d)


# speedup vs baseline: 1.2318x; 1.2318x over previous
import jax
import jax.numpy as jnp
from jax.experimental import pallas as pl

N, D, E, H = 2048, 1024, 8, 4096
BLK = 256
HB = 1024
NB = N // BLK
NH = H // HB


def _ffn_kernel(x_ref, disp_ref, comb_ref, wg_ref, wv_ref, wo_ref, scale_ref, out_ref):
    e = pl.program_id(0)
    h = pl.program_id(1)
    i = pl.program_id(2)

    rows = pl.ds(i * BLK, BLK)
    x = x_ref[rows, :]
    wg = wg_ref[0]
    wv = wv_ref[0]
    wo = wo_ref[0]

    gate = jax.lax.dot_general(x, wg, (((1,), (1,)), ((), ())),
                               preferred_element_type=jnp.float32)
    gate = gate * 0.5 * (1.0 + jax.lax.erf(gate * 0.7071067811865476))
    value = jax.lax.dot_general(x, wv, (((1,), (1,)), ((), ())),
                                preferred_element_type=jnp.float32)
    hidden = gate * value
    part = jax.lax.dot_general(hidden, wo, (((1,), (1,)), ((), ())),
                               preferred_element_type=jnp.float32)

    cols = jax.lax.broadcasted_iota(jnp.int32, (BLK, E), 1) == e
    dcol = jnp.sum(jnp.where(cols, disp_ref[rows, :], 0.0), axis=1, keepdims=True)
    ccol = jnp.sum(jnp.where(cols, comb_ref[rows, :], 0.0), axis=1, keepdims=True)
    srow = jax.lax.broadcasted_iota(jnp.int32, (1, E), 1) == e
    scale_e = jnp.sum(jnp.where(srow, scale_ref[...], 0.0))
    coef = jnp.where(dcol > 0, ccol * scale_e, 0.0)
    contrib = part * coef

    @pl.when(jnp.logical_and(e == 0, h == 0))
    def _init():
        out_ref[rows, :] = contrib

    @pl.when(jnp.logical_not(jnp.logical_and(e == 0, h == 0)))
    def _acc():
        out_ref[rows, :] += contrib


def kernel(tokens, dispatch_weights, combine_weights, Wg, Wv, Wo, scale):
    b, n, d = tokens.shape
    flat = tokens.reshape(n, d)
    disp = dispatch_weights.reshape(n, E)
    comb = combine_weights.reshape(n, E)

    out = pl.pallas_call(
        _ffn_kernel,
        grid=(E, NH, NB),
        in_specs=[
            pl.BlockSpec((N, D), lambda e, h, i: (0, 0)),
            pl.BlockSpec((N, E), lambda e, h, i: (0, 0)),
            pl.BlockSpec((N, E), lambda e, h, i: (0, 0)),
            pl.BlockSpec((1, HB, D), lambda e, h, i: (e, h, 0)),
            pl.BlockSpec((1, HB, D), lambda e, h, i: (e, h, 0)),
            pl.BlockSpec((1, D, HB), lambda e, h, i: (e, 0, h)),
            pl.BlockSpec((1, E), lambda e, h, i: (0, 0)),
        ],
        out_specs=pl.BlockSpec((N, D), lambda e, h, i: (0, 0)),
        out_shape=jax.ShapeDtypeStruct((N, D), jnp.float32),
    )(flat, disp, comb, Wg, Wv, Wo, scale.reshape(1, E))
    return out.reshape(b, n, d)
